# Initial kernel scaffold; baseline (speedup 1.0000x reference)
import jax
import jax.numpy as jnp
from jax.experimental import pallas as pl


def kernel(pos_fea, neg_fea, pos_classes, neg_classes, target_class, W1, b1, w2, b2, Wu1, bu1, wu2, bu2):
    raise NotImplementedError("write your pallas kernel here")


# speedup vs baseline: 3.3221x; 3.3221x over previous
import functools

import jax
import jax.numpy as jnp
import numpy as np
from jax.experimental import pallas as pl

_B, _K, _N, _D, _M, _H = 4, 16, 128, 256, 64, 32
_NTOP = 128
_UNARY_SCALES = [1.0, 0.5, 0.25, 0.125]


def _unary_kernel(pf_ref, neg_ref, Wu1_ref, bu1_ref, wu2_ref, bu2_ref, out_ref):
    f0 = pf_ref[0]
    f1 = neg_ref[0]
    h0 = jnp.dot(f0, Wu1_ref[...], preferred_element_type=jnp.float32)
    h1 = jnp.dot(f1, Wu1_ref[...], preferred_element_type=jnp.float32)
    pair = jax.nn.relu(h0[:, None, :] + h1[None, :, :] + bu1_ref[0][None, None, :])
    s = jnp.dot(
        pair.reshape(_N * _M, _H), wu2_ref[...], preferred_element_type=jnp.float32
    ).reshape(_N, _M) + bu2_ref[0, 0]
    out_ref[0, 0] = jnp.mean(s, axis=-1)


def _unary_energy(pf, neg_fea, Wu1, bu1, wu2, bu2):
    bk = _B * _K
    return pl.pallas_call(
        _unary_kernel,
        grid=(bk,),
        in_specs=[
            pl.BlockSpec((1, _N, _D), lambda i: (i, 0, 0)),
            pl.BlockSpec((1, _M, _D), lambda i: (i, 0, 0)),
            pl.BlockSpec((_D, _H), lambda i: (0, 0)),
            pl.BlockSpec((1, _H), lambda i: (0, 0)),
            pl.BlockSpec((_H, 1), lambda i: (0, 0)),
            pl.BlockSpec((1, 1), lambda i: (0, 0)),
        ],
        out_specs=pl.BlockSpec((1, 1, _N), lambda i: (i, 0, 0)),
        out_shape=jax.ShapeDtypeStruct((bk, 1, _N), jnp.float32),
    )(pf, neg_fea, Wu1, bu1.reshape(1, _H), wu2, bu2.reshape(1, 1)).reshape(bk, _N)


def _level_kernel(scale, P, f0_ref, f1_ref, pe0_ref, pe1_ref, ue0_ref, ue1_ref,
                  W1_ref, b1_ref, w2_ref, b2_ref, scores_ref, total_ref):
    h0 = jnp.dot(f0_ref[0], W1_ref[...], preferred_element_type=jnp.float32)
    h1 = jnp.dot(f1_ref[0], W1_ref[...], preferred_element_type=jnp.float32)
    pair = jax.nn.relu(h0[:, None, :] + h1[None, :, :] + b1_ref[0][None, None, :])
    s = jnp.dot(
        pair.reshape(P * P, _H), w2_ref[...], preferred_element_type=jnp.float32
    ).reshape(P, P) + b2_ref[0, 0]
    pe0 = pe0_ref[0, 0]
    pe1 = pe1_ref[0, 0]
    ue0 = ue0_ref[0, 0]
    ue1 = ue1_ref[0, 0]
    pw = (pe0[:, None] + pe1[None, :]) - s
    un = ue0[:, None] + ue1[None, :]
    scores_ref[0] = s
    total_ref[0] = pw + scale * un


def _level_scores(f0, f1, pe0, pe1, ue0, ue1, W1, b1, w2, b2, scale):
    B2, P, _ = f0.shape
    kern = functools.partial(_level_kernel, scale, P)
    return pl.pallas_call(
        kern,
        grid=(B2,),
        in_specs=[
            pl.BlockSpec((1, P, _D), lambda i: (i, 0, 0)),
            pl.BlockSpec((1, P, _D), lambda i: (i, 0, 0)),
            pl.BlockSpec((1, 1, P), lambda i: (i, 0, 0)),
            pl.BlockSpec((1, 1, P), lambda i: (i, 0, 0)),
            pl.BlockSpec((1, 1, P), lambda i: (i, 0, 0)),
            pl.BlockSpec((1, 1, P), lambda i: (i, 0, 0)),
            pl.BlockSpec((_D, _H), lambda i: (0, 0)),
            pl.BlockSpec((1, _H), lambda i: (0, 0)),
            pl.BlockSpec((_H, 1), lambda i: (0, 0)),
            pl.BlockSpec((1, 1), lambda i: (0, 0)),
        ],
        out_specs=[
            pl.BlockSpec((1, P, P), lambda i: (i, 0, 0)),
            pl.BlockSpec((1, P, P), lambda i: (i, 0, 0)),
        ],
        out_shape=[
            jax.ShapeDtypeStruct((B2, P, P), jnp.float32),
            jax.ShapeDtypeStruct((B2, P, P), jnp.float32),
        ],
    )(f0, f1, pe0[:, None, :], pe1[:, None, :], ue0[:, None, :], ue1[:, None, :],
      W1, b1.reshape(1, _H), w2, b2.reshape(1, 1))


def kernel(pos_fea, neg_fea, pos_classes, neg_classes, target_class,
           W1, b1, w2, b2, Wu1, bu1, wu2, bu2):
    pf = pos_fea.reshape(_B * _K, _N, _D)
    unary_energy = _unary_energy(pf, neg_fea, Wu1, bu1, wu2, bu2)

    subproblems = jnp.tile(jnp.arange(_N)[None, :, None], (_B * _K, 1, 1))
    pairwise_energy = jnp.zeros_like(unary_energy)
    tree_height = int(np.log2(_K))

    for i in range(tree_height):
        w = 2 ** i
        sub0, sub1 = subproblems[::2], subproblems[1::2]
        pe0, pe1 = pairwise_energy[::2], pairwise_energy[1::2]
        ue0, ue1 = unary_energy[::2], unary_energy[1::2]
        B2, P, _ = sub0.shape
        bags0 = (2 * jnp.arange(B2))[:, None] * w + jnp.arange(w)[None, :]
        bags1 = (2 * jnp.arange(B2) + 1)[:, None] * w + jnp.arange(w)[None, :]
        f0 = jnp.mean(pf[bags0[:, None, :], sub0], axis=2)
        f1 = jnp.mean(pf[bags1[:, None, :], sub1], axis=2)

        scores, total = _level_scores(
            f0, f1, pe0, pe1, ue0, ue1, W1, b1, w2, b2, _UNARY_SCALES[i]
        )

        total_flat = total.reshape(B2, P * P)
        topk = min(P * P, _NTOP) if i < tree_height - 1 else 1
        _, top_inds = jax.lax.top_k(-total_flat, topk)
        pI = top_inds // P
        qI = top_inds % P

        sub0_sel = jnp.take_along_axis(sub0, pI[..., None], axis=1)
        sub1_sel = jnp.take_along_axis(sub1, qI[..., None], axis=1)
        subproblems = jnp.concatenate([sub0_sel, sub1_sel], axis=-1)
        scores_sel = jnp.take_along_axis(scores.reshape(B2, P * P), top_inds, axis=1)
        pairwise_energy = (
            jnp.take_along_axis(pe0, pI, axis=1)
            + jnp.take_along_axis(pe1, qI, axis=1)
            - scores_sel
        )
        unary_energy = (
            jnp.take_along_axis(ue0, pI, axis=1)
            + jnp.take_along_axis(ue1, qI, axis=1)
        )

    tc = target_class.astype(jnp.float32)
    is_target = (pos_classes == tc[:, None, None]).astype(jnp.float32)
    return subproblems[:, 0], is_target


# speedup vs baseline: 3.4060x; 1.0252x over previous
import functools

import jax
import jax.numpy as jnp
import numpy as np
from jax.experimental import pallas as pl

_B, _K, _N, _D, _M, _H = 4, 16, 128, 256, 64, 32
_NTOP = 128
_UNARY_SCALES = [1.0, 0.5, 0.25, 0.125]


_GU = 4


def _unary_kernel(pf_ref, neg_ref, Wu1_ref, bu1_ref, wu2_ref, bu2_ref, out_ref):
    f0 = pf_ref[...]
    f1 = neg_ref[...]
    h0 = jnp.dot(
        f0.reshape(_GU * _N, _D), Wu1_ref[...], preferred_element_type=jnp.float32
    ).reshape(_GU, _N, _H)
    h1 = jnp.dot(
        f1.reshape(_GU * _M, _D), Wu1_ref[...], preferred_element_type=jnp.float32
    ).reshape(_GU, _M, _H)
    pair = jax.nn.relu(
        h0[:, :, None, :] + h1[:, None, :, :] + bu1_ref[0][None, None, None, :]
    )
    s = jnp.dot(
        pair.reshape(_GU * _N * _M, _H), wu2_ref[...],
        preferred_element_type=jnp.float32,
    ).reshape(_GU, _N, _M) + bu2_ref[0, 0]
    out_ref[:, 0, :] = jnp.mean(s, axis=-1)


def _unary_energy(pf, neg_fea, Wu1, bu1, wu2, bu2):
    bk = _B * _K
    return pl.pallas_call(
        _unary_kernel,
        grid=(bk // _GU,),
        in_specs=[
            pl.BlockSpec((_GU, _N, _D), lambda i: (i, 0, 0)),
            pl.BlockSpec((_GU, _M, _D), lambda i: (i, 0, 0)),
            pl.BlockSpec((_D, _H), lambda i: (0, 0)),
            pl.BlockSpec((1, _H), lambda i: (0, 0)),
            pl.BlockSpec((_H, 1), lambda i: (0, 0)),
            pl.BlockSpec((1, 1), lambda i: (0, 0)),
        ],
        out_specs=pl.BlockSpec((_GU, 1, _N), lambda i: (i, 0, 0)),
        out_shape=jax.ShapeDtypeStruct((bk, 1, _N), jnp.float32),
    )(pf, neg_fea, Wu1, bu1.reshape(1, _H), wu2, bu2.reshape(1, 1)).reshape(bk, _N)


def _level_kernel(scale, P, G, f0_ref, f1_ref, pe0_ref, pe1_ref, ue0_ref, ue1_ref,
                  W1_ref, b1_ref, w2_ref, b2_ref, scores_ref, total_ref):
    h0 = jnp.dot(
        f0_ref[...].reshape(G * P, _D), W1_ref[...],
        preferred_element_type=jnp.float32,
    ).reshape(G, P, _H)
    h1 = jnp.dot(
        f1_ref[...].reshape(G * P, _D), W1_ref[...],
        preferred_element_type=jnp.float32,
    ).reshape(G, P, _H)
    pair = jax.nn.relu(
        h0[:, :, None, :] + h1[:, None, :, :] + b1_ref[0][None, None, None, :]
    )
    s = jnp.dot(
        pair.reshape(G * P * P, _H), w2_ref[...], preferred_element_type=jnp.float32
    ).reshape(G, P, P) + b2_ref[0, 0]
    pe0 = pe0_ref[:, 0, :]
    pe1 = pe1_ref[:, 0, :]
    ue0 = ue0_ref[:, 0, :]
    ue1 = ue1_ref[:, 0, :]
    pw = (pe0[:, :, None] + pe1[:, None, :]) - s
    un = ue0[:, :, None] + ue1[:, None, :]
    scores_ref[...] = s
    total_ref[...] = pw + scale * un


def _level_scores(f0, f1, pe0, pe1, ue0, ue1, W1, b1, w2, b2, scale):
    B2, P, _ = f0.shape
    G = 2 if B2 >= 2 else 1
    kern = functools.partial(_level_kernel, scale, P, G)
    return pl.pallas_call(
        kern,
        grid=(B2 // G,),
        in_specs=[
            pl.BlockSpec((G, P, _D), lambda i: (i, 0, 0)),
            pl.BlockSpec((G, P, _D), lambda i: (i, 0, 0)),
            pl.BlockSpec((G, 1, P), lambda i: (i, 0, 0)),
            pl.BlockSpec((G, 1, P), lambda i: (i, 0, 0)),
            pl.BlockSpec((G, 1, P), lambda i: (i, 0, 0)),
            pl.BlockSpec((G, 1, P), lambda i: (i, 0, 0)),
            pl.BlockSpec((_D, _H), lambda i: (0, 0)),
            pl.BlockSpec((1, _H), lambda i: (0, 0)),
            pl.BlockSpec((_H, 1), lambda i: (0, 0)),
            pl.BlockSpec((1, 1), lambda i: (0, 0)),
        ],
        out_specs=[
            pl.BlockSpec((G, P, P), lambda i: (i, 0, 0)),
            pl.BlockSpec((G, P, P), lambda i: (i, 0, 0)),
        ],
        out_shape=[
            jax.ShapeDtypeStruct((B2, P, P), jnp.float32),
            jax.ShapeDtypeStruct((B2, P, P), jnp.float32),
        ],
    )(f0, f1, pe0[:, None, :], pe1[:, None, :], ue0[:, None, :], ue1[:, None, :],
      W1, b1.reshape(1, _H), w2, b2.reshape(1, 1))


def kernel(pos_fea, neg_fea, pos_classes, neg_classes, target_class,
           W1, b1, w2, b2, Wu1, bu1, wu2, bu2):
    pf = pos_fea.reshape(_B * _K, _N, _D)
    unary_energy = _unary_energy(pf, neg_fea, Wu1, bu1, wu2, bu2)

    subproblems = jnp.tile(jnp.arange(_N)[None, :, None], (_B * _K, 1, 1))
    pairwise_energy = jnp.zeros_like(unary_energy)
    tree_height = int(np.log2(_K))

    for i in range(tree_height):
        w = 2 ** i
        sub0, sub1 = subproblems[::2], subproblems[1::2]
        pe0, pe1 = pairwise_energy[::2], pairwise_energy[1::2]
        ue0, ue1 = unary_energy[::2], unary_energy[1::2]
        B2, P, _ = sub0.shape
        bags0 = (2 * jnp.arange(B2))[:, None] * w + jnp.arange(w)[None, :]
        bags1 = (2 * jnp.arange(B2) + 1)[:, None] * w + jnp.arange(w)[None, :]
        f0 = jnp.mean(pf[bags0[:, None, :], sub0], axis=2)
        f1 = jnp.mean(pf[bags1[:, None, :], sub1], axis=2)

        scores, total = _level_scores(
            f0, f1, pe0, pe1, ue0, ue1, W1, b1, w2, b2, _UNARY_SCALES[i]
        )

        total_flat = total.reshape(B2, P * P)
        topk = min(P * P, _NTOP) if i < tree_height - 1 else 1
        _, top_inds = jax.lax.top_k(-total_flat, topk)
        pI = top_inds // P
        qI = top_inds % P

        sub0_sel = jnp.take_along_axis(sub0, pI[..., None], axis=1)
        sub1_sel = jnp.take_along_axis(sub1, qI[..., None], axis=1)
        subproblems = jnp.concatenate([sub0_sel, sub1_sel], axis=-1)
        scores_sel = jnp.take_along_axis(scores.reshape(B2, P * P), top_inds, axis=1)
        pairwise_energy = (
            jnp.take_along_axis(pe0, pI, axis=1)
            + jnp.take_along_axis(pe1, qI, axis=1)
            - scores_sel
        )
        unary_energy = (
            jnp.take_along_axis(ue0, pI, axis=1)
            + jnp.take_along_axis(ue1, qI, axis=1)
        )

    tc = target_class.astype(jnp.float32)
    is_target = (pos_classes == tc[:, None, None]).astype(jnp.float32)
    return subproblems[:, 0], is_target
